# baseline (speedup 1.0000x reference)
import functools

import jax
import jax.numpy as jnp
from jax import lax
from jax.experimental import pallas as pl
from jax.experimental.pallas import tpu as pltpu
from jax.experimental.pallas import tpu_sc as plsc

D = 64
CHUNK = 256
NBUF = 4
LANES = 16


def _make_kernel(batch, hist, num_workers):
    total = batch * hist
    per_w = total // num_workers
    n_chunks = per_w // CHUNK
    n_groups = n_chunks // NBUF
    blocks_per_h = batch // CHUNK
    assert per_w * num_workers == total
    assert n_chunks * CHUNK == per_w
    assert n_groups * NBUF == n_chunks
    assert blocks_per_h * CHUNK == batch

    mesh = plsc.VectorSubcoreMesh(core_axis_name="c", subcore_axis_name="s")
    num_cores = mesh.num_cores

    @functools.partial(
        pl.kernel,
        out_type=jax.ShapeDtypeStruct((hist, D, batch), jnp.float32),
        mesh=mesh,
        scratch_types=[
            pltpu.VMEM((n_chunks, CHUNK), jnp.int32),
            pltpu.VMEM((NBUF, CHUNK, D), jnp.float32),
            pltpu.VMEM((D, CHUNK), jnp.float32),
        ] + [pltpu.SemaphoreType.DMA] * NBUF,
        compiler_params=pltpu.CompilerParams(
            use_tc_tiling_on_sc=False, needs_layout_passes=False
        ),
    )
    def emb_kernel(x_hbm, w_hbm, out_hbm, idx_v, rows_v, trans_v, *gsems):
        wid = lax.axis_index("s") * num_cores + lax.axis_index("c")
        chunk_base = wid * n_chunks

        pltpu.sync_copy(x_hbm.at[pl.ds(chunk_base, n_chunks)], idx_v)

        def start_gather(chunk, buf):
            pltpu.make_async_copy(
                w_hbm.at[idx_v.at[chunk]], rows_v.at[buf], gsems[buf]
            ).start()

        def wait_gather(chunk, buf):
            pltpu.make_async_copy(
                w_hbm.at[idx_v.at[chunk]], rows_v.at[buf], gsems[buf]
            ).wait()

        for b in range(NBUF):
            start_gather(b, b)

        iota = lax.iota(jnp.int32, LANES)

        def group_body(g, _):
            for b in range(NBUF):
                i = g * NBUF + b
                wait_gather(i, b)
                src = rows_v.at[b]

                def tr_body(k, _):
                    row_idx = iota + k * LANES
                    for c in range(D):
                        col_idx = jnp.full((LANES,), c, jnp.int32)
                        v = plsc.load_gather(src, [row_idx, col_idx])
                        trans_v[c, pl.ds(k * LANES, LANES)] = v

                lax.fori_loop(0, CHUNK // LANES, tr_body, None)

                j = chunk_base + i
                h = j // blocks_per_h
                b0 = (j % blocks_per_h) * CHUNK
                pltpu.sync_copy(
                    trans_v, out_hbm.at[h].at[:, pl.ds(b0, CHUNK)]
                )
                nxt = i + NBUF

                @pl.when(nxt < n_chunks)
                def _():
                    start_gather(nxt, b)

        lax.fori_loop(0, n_groups, group_body, None)

    return emb_kernel


def kernel(x, W):
    batch, hist = x.shape
    info = plsc.get_sparse_core_info()
    num_workers = info.num_cores * info.num_subcores
    x_rows = x.T.reshape(batch * hist // CHUNK, CHUNK).astype(jnp.int32)
    out_t = _make_kernel(batch, hist, num_workers)(x_rows, W)
    return out_t.transpose(2, 0, 1)


# speedup vs baseline: 1.1907x; 1.1907x over previous
import functools

import jax
import jax.numpy as jnp
from jax import lax
from jax.experimental import pallas as pl
from jax.experimental.pallas import tpu as pltpu
from jax.experimental.pallas import tpu_sc as plsc

D = 64
CHUNK = 256
NBUF = 4
LANES = 16


def _make_kernel(batch, hist, num_workers):
    total = batch * hist
    per_w = total // num_workers
    n_chunks = per_w // CHUNK
    n_groups = n_chunks // NBUF
    blocks_per_h = batch // CHUNK
    assert per_w * num_workers == total
    assert n_chunks * CHUNK == per_w
    assert n_groups * NBUF == n_chunks
    assert blocks_per_h * CHUNK == batch

    mesh = plsc.VectorSubcoreMesh(core_axis_name="c", subcore_axis_name="s")
    num_cores = mesh.num_cores

    @functools.partial(
        pl.kernel,
        out_type=jax.ShapeDtypeStruct((hist, D, batch), jnp.float32),
        mesh=mesh,
        scratch_types=[
            pltpu.VMEM((n_chunks, CHUNK), jnp.int32),
            pltpu.VMEM((NBUF, CHUNK, D), jnp.float32),
            pltpu.VMEM((D, CHUNK), jnp.float32),
        ] + [pltpu.SemaphoreType.DMA] * NBUF,
        compiler_params=pltpu.CompilerParams(
            use_tc_tiling_on_sc=False, needs_layout_passes=False
        ),
    )
    def emb_kernel(x_hbm, w_hbm, out_hbm, idx_v, rows_v, trans_v, *gsems):
        wid = lax.axis_index("s") * num_cores + lax.axis_index("c")
        chunk_base = wid * n_chunks

        pltpu.sync_copy(x_hbm.at[pl.ds(chunk_base, n_chunks)], idx_v)

        def start_gather(chunk, buf):
            pltpu.make_async_copy(
                w_hbm.at[idx_v.at[chunk]], rows_v.at[buf], gsems[buf]
            ).start()

        def wait_gather(chunk, buf):
            pltpu.make_async_copy(
                w_hbm.at[idx_v.at[chunk]], rows_v.at[buf], gsems[buf]
            ).wait()

        for b in range(NBUF):
            start_gather(b, b)

        iota = lax.iota(jnp.int32, LANES)

        def group_body(g, _):
            for b in range(NBUF):
                i = g * NBUF + b
                wait_gather(i, b)
                src = rows_v.at[b]

                @plsc.parallel_loop(0, CHUNK // LANES, unroll=4)
                def _(k):
                    row_idx = iota + k * LANES
                    for c in range(D):
                        col_idx = jnp.full((LANES,), c, jnp.int32)
                        v = plsc.load_gather(src, [row_idx, col_idx])
                        trans_v[c, pl.ds(k * LANES, LANES)] = v

                j = chunk_base + i
                h = j // blocks_per_h
                b0 = (j % blocks_per_h) * CHUNK
                pltpu.sync_copy(
                    trans_v, out_hbm.at[h].at[:, pl.ds(b0, CHUNK)]
                )
                nxt = i + NBUF

                @pl.when(nxt < n_chunks)
                def _():
                    start_gather(nxt, b)

        lax.fori_loop(0, n_groups, group_body, None)

    return emb_kernel


def kernel(x, W):
    batch, hist = x.shape
    info = plsc.get_sparse_core_info()
    num_workers = info.num_cores * info.num_subcores
    x_rows = x.T.reshape(batch * hist // CHUNK, CHUNK).astype(jnp.int32)
    out_t = _make_kernel(batch, hist, num_workers)(x_rows, W)
    return out_t.transpose(2, 0, 1)


# speedup vs baseline: 1.7544x; 1.4734x over previous
import functools

import jax
import jax.numpy as jnp
from jax import lax
from jax.experimental import pallas as pl
from jax.experimental.pallas import tpu as pltpu
from jax.experimental.pallas import tpu_sc as plsc

D = 64
CHUNK = 256
NBUF = 4
LANES = 16


def _make_kernel(batch, hist, num_workers):
    total = batch * hist
    per_w = total // num_workers
    n_chunks = per_w // CHUNK
    n_groups = n_chunks // NBUF
    blocks_per_h = batch // CHUNK
    assert per_w * num_workers == total
    assert n_chunks * CHUNK == per_w
    assert n_groups * NBUF == n_chunks
    assert blocks_per_h * CHUNK == batch

    mesh = plsc.VectorSubcoreMesh(core_axis_name="c", subcore_axis_name="s")
    num_cores = mesh.num_cores

    @functools.partial(
        pl.kernel,
        out_type=jax.ShapeDtypeStruct((hist, batch, D), jnp.float32),
        mesh=mesh,
        scratch_types=[
            pltpu.VMEM((n_chunks, CHUNK), jnp.int32),
            pltpu.VMEM((NBUF, CHUNK, D), jnp.float32),
            pltpu.VMEM((D, CHUNK), jnp.float32),
        ] + [pltpu.SemaphoreType.DMA] * NBUF,
        compiler_params=pltpu.CompilerParams(
            use_tc_tiling_on_sc=False, needs_layout_passes=False
        ),
    )
    def emb_kernel(x_hbm, w_hbm, out_hbm, idx_v, rows_v, trans_v, *gsems):
        wid = lax.axis_index("s") * num_cores + lax.axis_index("c")
        chunk_base = wid * n_chunks

        pltpu.sync_copy(x_hbm.at[pl.ds(chunk_base, n_chunks)], idx_v)

        def start_gather(chunk, buf):
            pltpu.make_async_copy(
                w_hbm.at[idx_v.at[chunk]], rows_v.at[buf], gsems[buf]
            ).start()

        def wait_gather(chunk, buf):
            pltpu.make_async_copy(
                w_hbm.at[idx_v.at[chunk]], rows_v.at[buf], gsems[buf]
            ).wait()

        for b in range(NBUF):
            start_gather(b, b)

        iota = lax.iota(jnp.int32, LANES)

        def group_body(g, _):
            for b in range(NBUF):
                i = g * NBUF + b
                wait_gather(i, b)
                src = rows_v.at[b]

                j = chunk_base + i
                h = j // blocks_per_h
                b0 = (j % blocks_per_h) * CHUNK
                pltpu.sync_copy(
                    src, out_hbm.at[h].at[pl.ds(b0, CHUNK), :]
                )
                nxt = i + NBUF

                @pl.when(nxt < n_chunks)
                def _():
                    start_gather(nxt, b)

        lax.fori_loop(0, n_groups, group_body, None)

    return emb_kernel


def kernel(x, W):
    batch, hist = x.shape
    info = plsc.get_sparse_core_info()
    num_workers = info.num_cores * info.num_subcores
    x_rows = x.T.reshape(batch * hist // CHUNK, CHUNK).astype(jnp.int32)
    out_t = _make_kernel(batch, hist, num_workers)(x_rows, W)
    return out_t.transpose(1, 0, 2)
